# baseline (speedup 1.0000x reference)
import functools
import jax
import jax.numpy as jnp
from jax import lax
from jax.experimental import pallas as pl
from jax.experimental.pallas import tpu as pltpu
from jax.experimental.pallas import tpu_sc as plsc

N = 10000
E = 320000
S = 160000
H = 100
D = 128
EPS = 1e-5

EP = 327680
SP = 163840
BE = 2048
BS = 2048

NW = 32
NC = 2


def _pad2(w, rows=D, cols=D):
    return jnp.zeros((rows, cols), jnp.float32).at[: w.shape[0], : w.shape[1]].set(w)


def _pad1(b, cols=D):
    return jnp.zeros((1, cols), jnp.float32).at[0, : b.shape[0]].set(b)


RPG = 64


@functools.partial(jax.jit, static_argnums=(2,))
def _sc_gather(table, idx2, B):
    KB = B // RPG // NW
    G = 10
    assert KB % G == 0

    @functools.partial(
        pl.kernel,
        out_type=jax.ShapeDtypeStruct((B, D), jnp.float32),
        mesh=plsc.VectorSubcoreMesh(core_axis_name="c", subcore_axis_name="s"),
        scratch_types=[
            pltpu.VMEM((KB, RPG), jnp.int32),
            pltpu.VMEM((G * RPG, D), jnp.float32),
            pltpu.SemaphoreType.DMA,
        ],
    )
    def k(table_hbm, idx_hbm, out_hbm, idx_v, rows_v, sem):
        wid = lax.axis_index("s") * NC + lax.axis_index("c")
        pltpu.sync_copy(idx_hbm.at[pl.ds(wid * KB, KB)], idx_v)

        def group(gi, carry):
            base = gi * G
            cps = [
                pltpu.async_copy(
                    table_hbm.at[idx_v.at[base + j]],
                    rows_v.at[pl.ds(j * RPG, RPG)],
                    sem,
                )
                for j in range(G)
            ]
            for cp in cps:
                cp.wait()
            pltpu.sync_copy(
                rows_v,
                out_hbm.at[pl.ds(wid * (KB * RPG) + base * RPG, G * RPG)],
            )
            return carry

        lax.fori_loop(0, KB // G, group, 0)

    return k(table, idx2)


def _edge_embed_body(ea_ref, w_ref, b_ref, o_ref):
    o_ref[...] = jnp.dot(ea_ref[...], w_ref[...],
                         preferred_element_type=jnp.float32) + b_ref[...]


def _edge_embed(edge_attr_p, Wp, bp):
    return pl.pallas_call(
        _edge_embed_body,
        grid=(EP // BE,),
        in_specs=[
            pl.BlockSpec((BE, 16), lambda i: (i, 0)),
            pl.BlockSpec((16, D), lambda i: (0, 0)),
            pl.BlockSpec((1, D), lambda i: (0, 0)),
        ],
        out_specs=pl.BlockSpec((BE, D), lambda i: (i, 0)),
        out_shape=jax.ShapeDtypeStruct((EP, D), jnp.float32),
    )(edge_attr_p, Wp, bp)


def _msg_body(g1_ref, ne_ref, vm_ref, wm_ref, bvec_ref, we1b_ref, msg_ref, ew1_ref):
    ne = ne_ref[...]
    pre = g1_ref[...] + jnp.dot(ne, wm_ref[...],
                                preferred_element_type=jnp.float32) + bvec_ref[...]
    msg_ref[...] = jnp.maximum(pre, 0.0) * vm_ref[...]
    ew1_ref[...] = jnp.dot(ne, we1b_ref[...], preferred_element_type=jnp.float32)


def _msg_stage(g1, ne, vmask, Wm_e_p, bvec_p, We1_b_p):
    return pl.pallas_call(
        _msg_body,
        grid=(SP // BS,),
        in_specs=[
            pl.BlockSpec((BS, D), lambda i: (i, 0)),
            pl.BlockSpec((BS, D), lambda i: (i, 0)),
            pl.BlockSpec((BS, 1), lambda i: (i, 0)),
            pl.BlockSpec((D, D), lambda i: (0, 0)),
            pl.BlockSpec((1, D), lambda i: (0, 0)),
            pl.BlockSpec((D, D), lambda i: (0, 0)),
        ],
        out_specs=[
            pl.BlockSpec((BS, D), lambda i: (i, 0)),
            pl.BlockSpec((BS, D), lambda i: (i, 0)),
        ],
        out_shape=[
            jax.ShapeDtypeStruct((SP, D), jnp.float32),
            jax.ShapeDtypeStruct((SP, D), jnp.float32),
        ],
    )(g1, ne, vmask, Wm_e_p, bvec_p, We1_b_p)


def _edge_mlp_body(g2_ref, g3_ref, ea_ref, w1_ref, b1_ref, w2_ref, b2_ref, o_ref):
    ea = ea_ref[...]
    hid = g2_ref[...] + g3_ref[...] + jnp.dot(
        ea, w1_ref[...], preferred_element_type=jnp.float32) + b1_ref[...]
    hid = jnp.maximum(hid, 0.0)
    upd = jnp.dot(hid, w2_ref[...], preferred_element_type=jnp.float32) + b2_ref[...]
    o_ref[...] = ea + 0.5 * upd


def _edge_mlp(g2, g3, eattr, We1_c_p, be1_p, We2_p, be2_p):
    return pl.pallas_call(
        _edge_mlp_body,
        grid=(EP // BE,),
        in_specs=[
            pl.BlockSpec((BE, D), lambda i: (i, 0)),
            pl.BlockSpec((BE, D), lambda i: (i, 0)),
            pl.BlockSpec((BE, D), lambda i: (i, 0)),
            pl.BlockSpec((D, D), lambda i: (0, 0)),
            pl.BlockSpec((1, D), lambda i: (0, 0)),
            pl.BlockSpec((D, D), lambda i: (0, 0)),
            pl.BlockSpec((1, D), lambda i: (0, 0)),
        ],
        out_specs=pl.BlockSpec((BE, D), lambda i: (i, 0)),
        out_shape=jax.ShapeDtypeStruct((EP, D), jnp.float32),
    )(g2, g3, eattr, We1_c_p, be1_p, We2_p, be2_p)


def _node_embed_body(x_ref, wn_ref, bn_ref, wm_ref, h_ref, hwm_ref):
    h = jnp.dot(x_ref[...], wn_ref[...], preferred_element_type=jnp.float32) + bn_ref[...]
    h_ref[...] = h
    hwm_ref[...] = jnp.dot(h, wm_ref[...], preferred_element_type=jnp.float32)


def _node_embed(x, W_node_p, b_node_p, Wm_h0_p):
    return pl.pallas_call(
        _node_embed_body,
        out_shape=[
            jax.ShapeDtypeStruct((N, D), jnp.float32),
            jax.ShapeDtypeStruct((N, D), jnp.float32),
        ],
    )(x, W_node_p, b_node_p, Wm_h0_p)


def _node_update_body(agg_ref, h_ref, wo_ref, bo_ref, g_ref, be_ref,
                      wa_ref, wb_ref, h2_ref, ta_ref, tb_ref):
    conv = jnp.dot(agg_ref[...], wo_ref[...],
                   preferred_element_type=jnp.float32) + bo_ref[...]
    mu = jnp.mean(conv, axis=0, keepdims=True)
    var = jnp.mean((conv - mu) * (conv - mu), axis=0, keepdims=True)
    bn = (conv - mu) * lax.rsqrt(var + EPS) * g_ref[...] + be_ref[...]
    h2 = (h_ref[...] + jnp.maximum(bn, 0.0)) * 0.5
    h2_ref[...] = h2
    ta_ref[...] = jnp.dot(h2, wa_ref[...], preferred_element_type=jnp.float32)
    tb_ref[...] = jnp.dot(h2, wb_ref[...], preferred_element_type=jnp.float32)


def _node_update(agg, h, Wo_p, bo_p, gamma_p, beta_p, Wa_p, Wb_p):
    return pl.pallas_call(
        _node_update_body,
        out_shape=[
            jax.ShapeDtypeStruct((N, D), jnp.float32),
            jax.ShapeDtypeStruct((N, D), jnp.float32),
            jax.ShapeDtypeStruct((N, D), jnp.float32),
        ],
    )(agg, h, Wo_p, bo_p, gamma_p, beta_p, Wa_p, Wb_p)


def _head_body(t_ref, b1_ref, w2_ref, b2_ref, w3_ref, b3_ref, o_ref):
    o1 = jnp.maximum(t_ref[...] + b1_ref[...], 0.0)
    o2 = jnp.maximum(jnp.dot(o1, w2_ref[...],
                             preferred_element_type=jnp.float32) + b2_ref[...], 0.0)
    o_ref[...] = jnp.dot(o2, w3_ref[...],
                         preferred_element_type=jnp.float32) + b3_ref[...]


def _head(t, bc1_p, Wc2_p, bc2_p, Wc3_p, bc3_p):
    return pl.pallas_call(
        _head_body,
        out_shape=jax.ShapeDtypeStruct((N, D), jnp.float32),
    )(t, bc1_p, Wc2_p, bc2_p, Wc3_p, bc3_p)


def _seg_meta_body(cnt_ref, ssrc_ref, sdst_ref, nsrc_ref, ndst_ref, vm_ref):
    cnt = cnt_ref[...]
    safe = jnp.maximum(cnt, 1.0)
    rows = jax.lax.broadcasted_iota(jnp.int32, cnt.shape, 0)
    cols = jax.lax.broadcasted_iota(jnp.int32, cnt.shape, 1)
    flat = rows * D + cols
    nsrc = jnp.clip(jnp.floor(ssrc_ref[...] / safe), 0.0, float(N - 1))
    ndst = jnp.clip(jnp.floor(sdst_ref[...] / safe), 0.0, float(N - 1))
    nsrc_ref[...] = nsrc.astype(jnp.int32)
    ndst_ref[...] = ndst.astype(jnp.int32)
    vm_ref[...] = jnp.where((cnt > 0.0) & (flat < S), 1.0, 0.0)


def _seg_meta(cnt2, ssrc2, sdst2):
    R = SP // D
    return pl.pallas_call(
        _seg_meta_body,
        out_shape=[
            jax.ShapeDtypeStruct((R, D), jnp.int32),
            jax.ShapeDtypeStruct((R, D), jnp.int32),
            jax.ShapeDtypeStruct((R, D), jnp.float32),
        ],
    )(cnt2, ssrc2, sdst2)


def _segsum(vals, sb, num):
    return jax.ops.segment_sum(vals, sb, num)


def kernel(x, edge_index, edge_attr, simp_edge_batch, W_node, b_node, W_edge, b_edge,
           Wm0, bm0, Wo0, bo0, gamma0, beta0, We1_0, be1_0, We2_0, be2_0,
           Wm1, bm1, Wo1, bo1, gamma1, beta1, We1_1, be1_1, We2_1, be2_1,
           Wc1, bc1, Wc2, bc2, Wc3, bc3):
    src = edge_index[0]
    dst = edge_index[1]
    sb = simp_edge_batch

    W_edge_p = _pad2(W_edge, 16, D)
    b_edge_p = _pad1(b_edge)
    W_node_p = _pad2(W_node)
    b_node_p = _pad1(b_node)
    layers = []
    for (Wm, bm, Wo, bo, gamma, beta, We1, be1, We2, be2) in (
            (Wm0, bm0, Wo0, bo0, gamma0, beta0, We1_0, be1_0, We2_0, be2_0),
            (Wm1, bm1, Wo1, bo1, gamma1, beta1, We1_1, be1_1, We2_1, be2_1)):
        layers.append(dict(
            Wm_h=_pad2(Wm[:H]), Wm_e=_pad2(Wm[H + 1:]),
            bvec=_pad1(bm + Wm[H]),
            Wo=_pad2(Wo), bo=_pad1(bo), gamma=_pad1(gamma), beta=_pad1(beta),
            We1_a=_pad2(We1[:H]), We1_b=_pad2(We1[H:2 * H]), We1_c=_pad2(We1[2 * H:]),
            be1=_pad1(be1), We2=_pad2(We2), be2=_pad1(be2),
        ))
    bc1_p = _pad1(bc1)
    Wc1_p = _pad2(Wc1)
    Wc2_p = _pad2(Wc2)
    bc2_p = _pad1(bc2)
    Wc3_p = _pad2(Wc3)
    bc3_p = _pad1(bc3)

    src2 = jnp.pad(src, (0, EP - E)).reshape(EP // RPG, RPG)
    sb2 = jnp.pad(sb, (0, EP - E)).reshape(EP // RPG, RPG)
    edge_attr_p = jnp.pad(edge_attr, ((0, EP - E), (0, 0)))

    ones = jnp.ones((E,), jnp.float32)
    cnt = _segsum(ones, sb, S)
    ssrc = _segsum(src.astype(jnp.float32), sb, S)
    sdst = _segsum(dst.astype(jnp.float32), sb, S)
    cnt2 = jnp.pad(cnt, (0, SP - S)).reshape(SP // D, D)
    ssrc2 = jnp.pad(ssrc, (0, SP - S)).reshape(SP // D, D)
    sdst2 = jnp.pad(sdst, (0, SP - S)).reshape(SP // D, D)
    nsrc2, ndst2, vm2 = _seg_meta(cnt2, ssrc2, sdst2)
    ndst_flat = ndst2.reshape(SP)
    vmask = vm2.reshape(SP, 1)
    nsrc2 = nsrc2.reshape(SP // RPG, RPG)

    h, hWm = _node_embed(x, W_node_p, b_node_p, layers[0]["Wm_h"])
    eattr = _edge_embed(edge_attr_p, W_edge_p, b_edge_p)

    for li, L in enumerate(layers):
        ne = _segsum(eattr[:E], sb, S)
        ne = jnp.pad(ne, ((0, SP - S), (0, 0)))
        g1 = _sc_gather(hWm, nsrc2, SP)
        msg, eW1 = _msg_stage(g1, ne, vmask, L["Wm_e"], L["bvec"], L["We1_b"])
        agg = _segsum(msg, ndst_flat, N)
        if li == 0:
            Wb = layers[1]["Wm_h"]
        else:
            Wb = Wc1_p
        h, hW1, hWm = _node_update(agg, h, L["Wo"], L["bo"], L["gamma"], L["beta"],
                                   L["We1_a"], Wb)
        g2 = _sc_gather(hW1, src2, EP)
        g3 = _sc_gather(eW1, sb2, EP)
        eattr = _edge_mlp(g2, g3, eattr, L["We1_c"], L["be1"], L["We2"], L["be2"])

    out = _head(hWm, bc1_p, Wc2_p, bc2_p, Wc3_p, bc3_p)
    return out[:, :2]


# speedup vs baseline: 1.3547x; 1.3547x over previous
import functools
import jax
import jax.numpy as jnp
from jax import lax
from jax.experimental import pallas as pl
from jax.experimental.pallas import tpu as pltpu
from jax.experimental.pallas import tpu_sc as plsc

N = 10000
E = 320000
S = 160000
H = 100
D = 128
EPS = 1e-5

EP = 327680
SP = 163840
BE = 2048
BS = 2048

NW = 32
NC = 2


def _pad2(w, rows=D, cols=D):
    return jnp.zeros((rows, cols), jnp.float32).at[: w.shape[0], : w.shape[1]].set(w)


def _pad1(b, cols=D):
    return jnp.zeros((1, cols), jnp.float32).at[0, : b.shape[0]].set(b)


RPG = 64


@functools.partial(jax.jit, static_argnums=(2,))
def _sc_gather(table, idx2, B):
    KB = B // RPG // NW
    G = 10
    assert KB % G == 0

    @functools.partial(
        pl.kernel,
        out_type=jax.ShapeDtypeStruct((B, D), jnp.float32),
        mesh=plsc.VectorSubcoreMesh(core_axis_name="c", subcore_axis_name="s"),
        scratch_types=[
            pltpu.VMEM((KB, RPG), jnp.int32),
            pltpu.VMEM((G * RPG, D), jnp.float32),
            pltpu.SemaphoreType.DMA,
        ],
    )
    def k(table_hbm, idx_hbm, out_hbm, idx_v, rows_v, sem):
        wid = lax.axis_index("s") * NC + lax.axis_index("c")
        pltpu.sync_copy(idx_hbm.at[pl.ds(wid * KB, KB)], idx_v)

        def group(gi, carry):
            base = gi * G
            cps = [
                pltpu.async_copy(
                    table_hbm.at[idx_v.at[base + j]],
                    rows_v.at[pl.ds(j * RPG, RPG)],
                    sem,
                )
                for j in range(G)
            ]
            for cp in cps:
                cp.wait()
            pltpu.sync_copy(
                rows_v,
                out_hbm.at[pl.ds(wid * (KB * RPG) + base * RPG, G * RPG)],
            )
            return carry

        lax.fori_loop(0, KB // G, group, 0)

    return k(table, idx2)


def _edge_embed_body(ea_ref, w_ref, b_ref, o_ref):
    o_ref[...] = jnp.dot(ea_ref[...], w_ref[...],
                         preferred_element_type=jnp.float32) + b_ref[...]


def _edge_embed(edge_attr_p, Wp, bp):
    return pl.pallas_call(
        _edge_embed_body,
        grid=(EP // BE,),
        in_specs=[
            pl.BlockSpec((BE, 16), lambda i: (i, 0)),
            pl.BlockSpec((16, D), lambda i: (0, 0)),
            pl.BlockSpec((1, D), lambda i: (0, 0)),
        ],
        out_specs=pl.BlockSpec((BE, D), lambda i: (i, 0)),
        out_shape=jax.ShapeDtypeStruct((EP, D), jnp.float32),
    )(edge_attr_p, Wp, bp)


def _msg_body(g1_ref, ne_ref, vm_ref, wm_ref, bvec_ref, we1b_ref, msg_ref, ew1_ref):
    ne = ne_ref[...]
    pre = g1_ref[...] + jnp.dot(ne, wm_ref[...],
                                preferred_element_type=jnp.float32) + bvec_ref[...]
    msg_ref[...] = jnp.maximum(pre, 0.0) * vm_ref[...]
    ew1_ref[...] = jnp.dot(ne, we1b_ref[...], preferred_element_type=jnp.float32)


def _msg_stage(g1, ne, vmask, Wm_e_p, bvec_p, We1_b_p):
    return pl.pallas_call(
        _msg_body,
        grid=(SP // BS,),
        in_specs=[
            pl.BlockSpec((BS, D), lambda i: (i, 0)),
            pl.BlockSpec((BS, D), lambda i: (i, 0)),
            pl.BlockSpec((BS, 1), lambda i: (i, 0)),
            pl.BlockSpec((D, D), lambda i: (0, 0)),
            pl.BlockSpec((1, D), lambda i: (0, 0)),
            pl.BlockSpec((D, D), lambda i: (0, 0)),
        ],
        out_specs=[
            pl.BlockSpec((BS, D), lambda i: (i, 0)),
            pl.BlockSpec((BS, D), lambda i: (i, 0)),
        ],
        out_shape=[
            jax.ShapeDtypeStruct((SP, D), jnp.float32),
            jax.ShapeDtypeStruct((SP, D), jnp.float32),
        ],
    )(g1, ne, vmask, Wm_e_p, bvec_p, We1_b_p)


def _edge_mlp_body(g2_ref, g3_ref, ea_ref, w1_ref, b1_ref, w2_ref, b2_ref, o_ref):
    ea = ea_ref[...]
    hid = g2_ref[...] + g3_ref[...] + jnp.dot(
        ea, w1_ref[...], preferred_element_type=jnp.float32) + b1_ref[...]
    hid = jnp.maximum(hid, 0.0)
    upd = jnp.dot(hid, w2_ref[...], preferred_element_type=jnp.float32) + b2_ref[...]
    o_ref[...] = ea + 0.5 * upd


def _edge_mlp(g2, g3, eattr, We1_c_p, be1_p, We2_p, be2_p):
    return pl.pallas_call(
        _edge_mlp_body,
        grid=(EP // BE,),
        in_specs=[
            pl.BlockSpec((BE, D), lambda i: (i, 0)),
            pl.BlockSpec((BE, D), lambda i: (i, 0)),
            pl.BlockSpec((BE, D), lambda i: (i, 0)),
            pl.BlockSpec((D, D), lambda i: (0, 0)),
            pl.BlockSpec((1, D), lambda i: (0, 0)),
            pl.BlockSpec((D, D), lambda i: (0, 0)),
            pl.BlockSpec((1, D), lambda i: (0, 0)),
        ],
        out_specs=pl.BlockSpec((BE, D), lambda i: (i, 0)),
        out_shape=jax.ShapeDtypeStruct((EP, D), jnp.float32),
    )(g2, g3, eattr, We1_c_p, be1_p, We2_p, be2_p)


def _node_embed_body(x_ref, wn_ref, bn_ref, wm_ref, h_ref, hwm_ref):
    h = jnp.dot(x_ref[...], wn_ref[...], preferred_element_type=jnp.float32) + bn_ref[...]
    h_ref[...] = h
    hwm_ref[...] = jnp.dot(h, wm_ref[...], preferred_element_type=jnp.float32)


def _node_embed(x, W_node_p, b_node_p, Wm_h0_p):
    return pl.pallas_call(
        _node_embed_body,
        out_shape=[
            jax.ShapeDtypeStruct((N, D), jnp.float32),
            jax.ShapeDtypeStruct((N, D), jnp.float32),
        ],
    )(x, W_node_p, b_node_p, Wm_h0_p)


def _node_update_body(agg_ref, h_ref, wo_ref, bo_ref, g_ref, be_ref,
                      wa_ref, wb_ref, h2_ref, ta_ref, tb_ref):
    conv = jnp.dot(agg_ref[...], wo_ref[...],
                   preferred_element_type=jnp.float32) + bo_ref[...]
    mu = jnp.mean(conv, axis=0, keepdims=True)
    var = jnp.mean((conv - mu) * (conv - mu), axis=0, keepdims=True)
    bn = (conv - mu) * lax.rsqrt(var + EPS) * g_ref[...] + be_ref[...]
    h2 = (h_ref[...] + jnp.maximum(bn, 0.0)) * 0.5
    h2_ref[...] = h2
    ta_ref[...] = jnp.dot(h2, wa_ref[...], preferred_element_type=jnp.float32)
    tb_ref[...] = jnp.dot(h2, wb_ref[...], preferred_element_type=jnp.float32)


def _node_update(agg, h, Wo_p, bo_p, gamma_p, beta_p, Wa_p, Wb_p):
    return pl.pallas_call(
        _node_update_body,
        out_shape=[
            jax.ShapeDtypeStruct((N, D), jnp.float32),
            jax.ShapeDtypeStruct((N, D), jnp.float32),
            jax.ShapeDtypeStruct((N, D), jnp.float32),
        ],
    )(agg, h, Wo_p, bo_p, gamma_p, beta_p, Wa_p, Wb_p)


def _head_body(t_ref, b1_ref, w2_ref, b2_ref, w3_ref, b3_ref, o_ref):
    o1 = jnp.maximum(t_ref[...] + b1_ref[...], 0.0)
    o2 = jnp.maximum(jnp.dot(o1, w2_ref[...],
                             preferred_element_type=jnp.float32) + b2_ref[...], 0.0)
    o_ref[...] = jnp.dot(o2, w3_ref[...],
                         preferred_element_type=jnp.float32) + b3_ref[...]


def _head(t, bc1_p, Wc2_p, bc2_p, Wc3_p, bc3_p):
    return pl.pallas_call(
        _head_body,
        out_shape=jax.ShapeDtypeStruct((N, D), jnp.float32),
    )(t, bc1_p, Wc2_p, bc2_p, Wc3_p, bc3_p)


def _seg_meta_body(cnt_ref, ssrc_ref, sdst_ref, nsrc_ref, ndst_ref, vm_ref):
    cnt = cnt_ref[...]
    safe = jnp.maximum(cnt, 1.0)
    rows = jax.lax.broadcasted_iota(jnp.int32, cnt.shape, 0)
    cols = jax.lax.broadcasted_iota(jnp.int32, cnt.shape, 1)
    flat = rows * D + cols
    nsrc = jnp.clip(jnp.floor(ssrc_ref[...] / safe), 0.0, float(N - 1))
    ndst = jnp.clip(jnp.floor(sdst_ref[...] / safe), 0.0, float(N - 1))
    nsrc_ref[...] = nsrc.astype(jnp.int32)
    ndst_ref[...] = ndst.astype(jnp.int32)
    vm_ref[...] = jnp.where((cnt > 0.0) & (flat < S), 1.0, 0.0)


def _seg_meta(cnt2, ssrc2, sdst2):
    R = SP // D
    return pl.pallas_call(
        _seg_meta_body,
        out_shape=[
            jax.ShapeDtypeStruct((R, D), jnp.int32),
            jax.ShapeDtypeStruct((R, D), jnp.int32),
            jax.ShapeDtypeStruct((R, D), jnp.float32),
        ],
    )(cnt2, ssrc2, sdst2)


def _segsum(vals, sb, num):
    return jax.ops.segment_sum(vals, sb, num)


def kernel(x, edge_index, edge_attr, simp_edge_batch, W_node, b_node, W_edge, b_edge,
           Wm0, bm0, Wo0, bo0, gamma0, beta0, We1_0, be1_0, We2_0, be2_0,
           Wm1, bm1, Wo1, bo1, gamma1, beta1, We1_1, be1_1, We2_1, be2_1,
           Wc1, bc1, Wc2, bc2, Wc3, bc3):
    src = edge_index[0]
    dst = edge_index[1]
    sb = simp_edge_batch

    W_edge_p = _pad2(W_edge, 16, D)
    b_edge_p = _pad1(b_edge)
    W_node_p = _pad2(W_node)
    b_node_p = _pad1(b_node)
    layers = []
    for (Wm, bm, Wo, bo, gamma, beta, We1, be1, We2, be2) in (
            (Wm0, bm0, Wo0, bo0, gamma0, beta0, We1_0, be1_0, We2_0, be2_0),
            (Wm1, bm1, Wo1, bo1, gamma1, beta1, We1_1, be1_1, We2_1, be2_1)):
        layers.append(dict(
            Wm_h=_pad2(Wm[:H]), Wm_e=_pad2(Wm[H + 1:]),
            bvec=_pad1(bm + Wm[H]),
            Wo=_pad2(Wo), bo=_pad1(bo), gamma=_pad1(gamma), beta=_pad1(beta),
            We1_a=_pad2(We1[:H]), We1_b=_pad2(We1[H:2 * H]), We1_c=_pad2(We1[2 * H:]),
            be1=_pad1(be1), We2=_pad2(We2), be2=_pad1(be2),
        ))
    bc1_p = _pad1(bc1)
    Wc1_p = _pad2(Wc1)
    Wc2_p = _pad2(Wc2)
    bc2_p = _pad1(bc2)
    Wc3_p = _pad2(Wc3)
    bc3_p = _pad1(bc3)

    src2 = jnp.pad(src, (0, EP - E)).reshape(EP // RPG, RPG)
    sb2 = jnp.pad(sb, (0, EP - E)).reshape(EP // RPG, RPG)
    edge_attr_p = jnp.pad(edge_attr, ((0, EP - E), (0, 0)))

    ones = jnp.ones((E,), jnp.float32)
    cnt = _segsum(ones, sb, S)
    ssrc = _segsum(src.astype(jnp.float32), sb, S)
    sdst = _segsum(dst.astype(jnp.float32), sb, S)
    cnt2 = jnp.pad(cnt, (0, SP - S)).reshape(SP // D, D)
    ssrc2 = jnp.pad(ssrc, (0, SP - S)).reshape(SP // D, D)
    sdst2 = jnp.pad(sdst, (0, SP - S)).reshape(SP // D, D)
    nsrc2, ndst2, vm2 = _seg_meta(cnt2, ssrc2, sdst2)
    ndst_flat = ndst2.reshape(SP)
    vmask = vm2.reshape(SP, 1)
    nsrc_flat = nsrc2.reshape(SP)

    h, hWm = _node_embed(x, W_node_p, b_node_p, layers[0]["Wm_h"])
    eattr = _edge_embed(edge_attr_p, W_edge_p, b_edge_p)

    for li, L in enumerate(layers):
        ne = _segsum(eattr[:E], sb, S)
        ne = jnp.pad(ne, ((0, SP - S), (0, 0)))
        g1 = hWm[nsrc_flat]
        msg, eW1 = _msg_stage(g1, ne, vmask, L["Wm_e"], L["bvec"], L["We1_b"])
        agg = _segsum(msg, ndst_flat, N)
        if li == 0:
            Wb = layers[1]["Wm_h"]
        else:
            Wb = Wc1_p
        h, hW1, hWm = _node_update(agg, h, L["Wo"], L["bo"], L["gamma"], L["beta"],
                                   L["We1_a"], Wb)
        g2 = _sc_gather(hW1, src2, EP)
        g3 = _sc_gather(eW1, sb2, EP)
        eattr = _edge_mlp(g2, g3, eattr, L["We1_c"], L["be1"], L["We2"], L["be2"])

    out = _head(hWm, bc1_p, Wc2_p, bc2_p, Wc3_p, bc3_p)
    return out[:, :2]


# speedup vs baseline: 1.4919x; 1.1012x over previous
import functools
import jax
import jax.numpy as jnp
from jax import lax
from jax.experimental import pallas as pl
from jax.experimental.pallas import tpu as pltpu
from jax.experimental.pallas import tpu_sc as plsc

N = 10000
E = 320000
S = 160000
H = 100
D = 128
EPS = 1e-5

EP = 327680
SP = 163840
BE = 2048
BS = 2048

NW = 32
NC = 2


def _pad2(w, rows=D, cols=D):
    return jnp.zeros((rows, cols), jnp.float32).at[: w.shape[0], : w.shape[1]].set(w)


def _pad1(b, cols=D):
    return jnp.zeros((1, cols), jnp.float32).at[0, : b.shape[0]].set(b)


RPG = 64


@functools.partial(jax.jit, static_argnums=(2,))
def _sc_gather(table, idx2, B):
    KB = B // RPG // NW
    G = 10
    assert KB % G == 0

    @functools.partial(
        pl.kernel,
        out_type=jax.ShapeDtypeStruct((B, D), jnp.float32),
        mesh=plsc.VectorSubcoreMesh(core_axis_name="c", subcore_axis_name="s"),
        scratch_types=[
            pltpu.VMEM((KB, RPG), jnp.int32),
            pltpu.VMEM((G * RPG, D), jnp.float32),
            pltpu.SemaphoreType.DMA,
        ],
    )
    def k(table_hbm, idx_hbm, out_hbm, idx_v, rows_v, sem):
        wid = lax.axis_index("s") * NC + lax.axis_index("c")
        pltpu.sync_copy(idx_hbm.at[pl.ds(wid * KB, KB)], idx_v)

        def group(gi, carry):
            base = gi * G
            cps = [
                pltpu.async_copy(
                    table_hbm.at[idx_v.at[base + j]],
                    rows_v.at[pl.ds(j * RPG, RPG)],
                    sem,
                )
                for j in range(G)
            ]
            for cp in cps:
                cp.wait()
            pltpu.sync_copy(
                rows_v,
                out_hbm.at[pl.ds(wid * (KB * RPG) + base * RPG, G * RPG)],
            )
            return carry

        lax.fori_loop(0, KB // G, group, 0)

    return k(table, idx2)


def _sc_segsum_ne(rows, sb_seg):
    R = EP // NW
    CH = 512

    @functools.partial(
        pl.kernel,
        out_type=[
            jax.ShapeDtypeStruct((SP, D), jnp.float32),
            jax.ShapeDtypeStruct((2 * NW * D,), jnp.float32),
            jax.ShapeDtypeStruct((NW * 16,), jnp.int32),
        ],
        mesh=plsc.VectorSubcoreMesh(core_axis_name="c", subcore_axis_name="s"),
        scratch_types=[
            pltpu.VMEM((R + 16,), jnp.int32),
            pltpu.VMEM((CH, D), jnp.float32),
            pltpu.VMEM((128, D), jnp.float32),
            pltpu.VMEM((128,), jnp.int32),
            pltpu.VMEM((2 * D,), jnp.float32),
            pltpu.VMEM((16,), jnp.int32),
            pltpu.SemaphoreType.DMA,
        ],
    )
    def k(sb_hbm, rows_hbm, ne_hbm, side_hbm, sideid_hbm,
          sb_v, rv, stv, sti, sdv, sdi, sem):
        wid = lax.axis_index("s") * NC + lax.axis_index("c")
        base = wid * R
        pltpu.sync_copy(sb_hbm.at[pl.ds(base, R + 16)], sb_v)

        dump = jnp.full((16,), SP - 1, jnp.int32)
        for t in range(8):
            sti[pl.ds(t * 16, 16)] = dump
        sdi[...] = dump
        zero = jnp.zeros((16,), jnp.float32)
        for kk in range(8):
            sdv[pl.ds(kk * 16, 16)] = zero

        def chunk_body(c, carry):
            pltpu.sync_copy(
                rows_hbm.at[pl.ds(base + c * CH, CH)], rv)

            def row_body(i, carry2):
                (a0, a1, a2, a3, a4, a5, a6, a7, scur, nstage, nflush) = carry2
                acc = (a0, a1, a2, a3, a4, a5, a6, a7)
                s = sb_v[pl.ds(c * CH + i, 16)][0]
                row = [rv[i, pl.ds(kk * 16, 16)] for kk in range(8)]
                is_new = s != scur
                lane = lax.iota(jnp.int32, 16)

                @pl.when(is_new & (nflush == 0))
                def _():
                    for kk in range(8):
                        sdv[pl.ds(kk * 16, 16)] = acc[kk]
                    sdi[...] = jnp.where(lane == 0, scur, sdi[...])

                @pl.when(is_new & (nflush > 0))
                def _():
                    stv_row = stv.at[nstage]
                    for kk in range(8):
                        stv_row[pl.ds(kk * 16, 16)] = acc[kk]
                    sl = (nstage // 16) * 16
                    old = sti[pl.ds(sl, 16)]
                    sti[pl.ds(sl, 16)] = jnp.where(lane == nstage % 16, scur, old)

                @pl.when(is_new & (nflush > 0) & (nstage == 127))
                def _():
                    pltpu.async_copy(stv, ne_hbm.at[sti], sem).wait()
                    for t in range(8):
                        sti[pl.ds(t * 16, 16)] = dump

                staged = is_new & (nflush > 0)
                nstage_n = jnp.where(
                    staged, jnp.where(nstage == 127, 0, nstage + 1), nstage)
                nflush_n = jnp.where(is_new, nflush + 1, nflush)
                scur_n = jnp.where(is_new, s, scur)
                acc_n = [jnp.where(is_new, row[kk], acc[kk] + row[kk])
                         for kk in range(8)]
                return tuple(acc_n + [scur_n, nstage_n, nflush_n])

            return lax.fori_loop(0, CH, row_body, carry)

        z = jnp.zeros((16,), jnp.float32)
        init = (z, z, z, z, z, z, z, z, sb_v[pl.ds(0, 16)][0],
                jnp.int32(0), jnp.int32(0))
        fin = lax.fori_loop(0, R // CH, chunk_body, init)
        accf = fin[:8]
        scurf = fin[8]
        for kk in range(8):
            sdv[pl.ds(D + kk * 16, 16)] = accf[kk]
        lane = lax.iota(jnp.int32, 16)
        sdi[...] = jnp.where(lane == 1, scurf, sdi[...])
        pltpu.async_copy(stv, ne_hbm.at[sti], sem).wait()
        pltpu.sync_copy(sdv, side_hbm.at[pl.ds(wid * 2 * D, 2 * D)])
        pltpu.sync_copy(sdi, sideid_hbm.at[pl.ds(wid * 16, 16)])

    return k(sb_seg, rows)


def _edge_embed_body(ea_ref, w_ref, b_ref, o_ref):
    o_ref[...] = jnp.dot(ea_ref[...], w_ref[...],
                         preferred_element_type=jnp.float32) + b_ref[...]


def _edge_embed(edge_attr_p, Wp, bp):
    return pl.pallas_call(
        _edge_embed_body,
        grid=(EP // BE,),
        in_specs=[
            pl.BlockSpec((BE, 16), lambda i: (i, 0)),
            pl.BlockSpec((16, D), lambda i: (0, 0)),
            pl.BlockSpec((1, D), lambda i: (0, 0)),
        ],
        out_specs=pl.BlockSpec((BE, D), lambda i: (i, 0)),
        out_shape=jax.ShapeDtypeStruct((EP, D), jnp.float32),
    )(edge_attr_p, Wp, bp)


def _msg_body(g1_ref, ne_ref, vm_ref, sd_ref, sid_ref,
              wm_ref, bvec_ref, we1b_ref, msg_ref, ew1_ref):
    i = pl.program_id(0)
    rows = jax.lax.broadcasted_iota(jnp.int32, (BS, 2 * NW), 0) + i * BS
    onehot = (rows == sid_ref[...]).astype(jnp.float32)
    corr = jnp.dot(onehot, sd_ref[...], preferred_element_type=jnp.float32)
    bmask = jnp.sum(onehot, axis=1, keepdims=True)
    ne = jnp.where(bmask > 0.0, corr, ne_ref[...])
    pre = g1_ref[...] + jnp.dot(ne, wm_ref[...],
                                preferred_element_type=jnp.float32) + bvec_ref[...]
    msg_ref[...] = jnp.maximum(pre, 0.0) * vm_ref[...]
    ew1_ref[...] = jnp.dot(ne, we1b_ref[...], preferred_element_type=jnp.float32)


def _msg_stage(g1, ne, vmask, sides, sideids, Wm_e_p, bvec_p, We1_b_p):
    return pl.pallas_call(
        _msg_body,
        grid=(SP // BS,),
        in_specs=[
            pl.BlockSpec((BS, D), lambda i: (i, 0)),
            pl.BlockSpec((BS, D), lambda i: (i, 0)),
            pl.BlockSpec((BS, 1), lambda i: (i, 0)),
            pl.BlockSpec((2 * NW, D), lambda i: (0, 0)),
            pl.BlockSpec((1, 2 * NW), lambda i: (0, 0)),
            pl.BlockSpec((D, D), lambda i: (0, 0)),
            pl.BlockSpec((1, D), lambda i: (0, 0)),
            pl.BlockSpec((D, D), lambda i: (0, 0)),
        ],
        out_specs=[
            pl.BlockSpec((BS, D), lambda i: (i, 0)),
            pl.BlockSpec((BS, D), lambda i: (i, 0)),
        ],
        out_shape=[
            jax.ShapeDtypeStruct((SP, D), jnp.float32),
            jax.ShapeDtypeStruct((SP, D), jnp.float32),
        ],
    )(g1, ne, vmask, sides, sideids, Wm_e_p, bvec_p, We1_b_p)


def _edge_mlp_body(g2_ref, g3_ref, ea_ref, w1_ref, b1_ref, w2_ref, b2_ref, o_ref):
    ea = ea_ref[...]
    hid = g2_ref[...] + g3_ref[...] + jnp.dot(
        ea, w1_ref[...], preferred_element_type=jnp.float32) + b1_ref[...]
    hid = jnp.maximum(hid, 0.0)
    upd = jnp.dot(hid, w2_ref[...], preferred_element_type=jnp.float32) + b2_ref[...]
    o_ref[...] = ea + 0.5 * upd


def _edge_mlp(g2, g3, eattr, We1_c_p, be1_p, We2_p, be2_p):
    return pl.pallas_call(
        _edge_mlp_body,
        grid=(EP // BE,),
        in_specs=[
            pl.BlockSpec((BE, D), lambda i: (i, 0)),
            pl.BlockSpec((BE, D), lambda i: (i, 0)),
            pl.BlockSpec((BE, D), lambda i: (i, 0)),
            pl.BlockSpec((D, D), lambda i: (0, 0)),
            pl.BlockSpec((1, D), lambda i: (0, 0)),
            pl.BlockSpec((D, D), lambda i: (0, 0)),
            pl.BlockSpec((1, D), lambda i: (0, 0)),
        ],
        out_specs=pl.BlockSpec((BE, D), lambda i: (i, 0)),
        out_shape=jax.ShapeDtypeStruct((EP, D), jnp.float32),
    )(g2, g3, eattr, We1_c_p, be1_p, We2_p, be2_p)


def _node_embed_body(x_ref, wn_ref, bn_ref, wm_ref, h_ref, hwm_ref):
    h = jnp.dot(x_ref[...], wn_ref[...], preferred_element_type=jnp.float32) + bn_ref[...]
    h_ref[...] = h
    hwm_ref[...] = jnp.dot(h, wm_ref[...], preferred_element_type=jnp.float32)


def _node_embed(x, W_node_p, b_node_p, Wm_h0_p):
    return pl.pallas_call(
        _node_embed_body,
        out_shape=[
            jax.ShapeDtypeStruct((N, D), jnp.float32),
            jax.ShapeDtypeStruct((N, D), jnp.float32),
        ],
    )(x, W_node_p, b_node_p, Wm_h0_p)


def _node_update_body(agg_ref, h_ref, wo_ref, bo_ref, g_ref, be_ref,
                      wa_ref, wb_ref, h2_ref, ta_ref, tb_ref):
    conv = jnp.dot(agg_ref[...], wo_ref[...],
                   preferred_element_type=jnp.float32) + bo_ref[...]
    mu = jnp.mean(conv, axis=0, keepdims=True)
    var = jnp.mean((conv - mu) * (conv - mu), axis=0, keepdims=True)
    bn = (conv - mu) * lax.rsqrt(var + EPS) * g_ref[...] + be_ref[...]
    h2 = (h_ref[...] + jnp.maximum(bn, 0.0)) * 0.5
    h2_ref[...] = h2
    ta_ref[...] = jnp.dot(h2, wa_ref[...], preferred_element_type=jnp.float32)
    tb_ref[...] = jnp.dot(h2, wb_ref[...], preferred_element_type=jnp.float32)


def _node_update(agg, h, Wo_p, bo_p, gamma_p, beta_p, Wa_p, Wb_p):
    return pl.pallas_call(
        _node_update_body,
        out_shape=[
            jax.ShapeDtypeStruct((N, D), jnp.float32),
            jax.ShapeDtypeStruct((N, D), jnp.float32),
            jax.ShapeDtypeStruct((N, D), jnp.float32),
        ],
    )(agg, h, Wo_p, bo_p, gamma_p, beta_p, Wa_p, Wb_p)


def _head_body(t_ref, b1_ref, w2_ref, b2_ref, w3_ref, b3_ref, o_ref):
    o1 = jnp.maximum(t_ref[...] + b1_ref[...], 0.0)
    o2 = jnp.maximum(jnp.dot(o1, w2_ref[...],
                             preferred_element_type=jnp.float32) + b2_ref[...], 0.0)
    o_ref[...] = jnp.dot(o2, w3_ref[...],
                         preferred_element_type=jnp.float32) + b3_ref[...]


def _head(t, bc1_p, Wc2_p, bc2_p, Wc3_p, bc3_p):
    return pl.pallas_call(
        _head_body,
        out_shape=jax.ShapeDtypeStruct((N, D), jnp.float32),
    )(t, bc1_p, Wc2_p, bc2_p, Wc3_p, bc3_p)


def _seg_meta_body(cnt_ref, ssrc_ref, sdst_ref, nsrc_ref, ndst_ref, vm_ref):
    cnt = cnt_ref[...]
    safe = jnp.maximum(cnt, 1.0)
    rows = jax.lax.broadcasted_iota(jnp.int32, cnt.shape, 0)
    cols = jax.lax.broadcasted_iota(jnp.int32, cnt.shape, 1)
    flat = rows * D + cols
    nsrc = jnp.clip(jnp.floor(ssrc_ref[...] / safe), 0.0, float(N - 1))
    ndst = jnp.clip(jnp.floor(sdst_ref[...] / safe), 0.0, float(N - 1))
    nsrc_ref[...] = nsrc.astype(jnp.int32)
    ndst_ref[...] = ndst.astype(jnp.int32)
    vm_ref[...] = jnp.where((cnt > 0.0) & (flat < S), 1.0, 0.0)


def _seg_meta(cnt2, ssrc2, sdst2):
    R = SP // D
    return pl.pallas_call(
        _seg_meta_body,
        out_shape=[
            jax.ShapeDtypeStruct((R, D), jnp.int32),
            jax.ShapeDtypeStruct((R, D), jnp.int32),
            jax.ShapeDtypeStruct((R, D), jnp.float32),
        ],
    )(cnt2, ssrc2, sdst2)


def _segsum(vals, sb, num):
    return jax.ops.segment_sum(vals, sb, num)


def kernel(x, edge_index, edge_attr, simp_edge_batch, W_node, b_node, W_edge, b_edge,
           Wm0, bm0, Wo0, bo0, gamma0, beta0, We1_0, be1_0, We2_0, be2_0,
           Wm1, bm1, Wo1, bo1, gamma1, beta1, We1_1, be1_1, We2_1, be2_1,
           Wc1, bc1, Wc2, bc2, Wc3, bc3):
    src = edge_index[0]
    dst = edge_index[1]
    sb = simp_edge_batch

    W_edge_p = _pad2(W_edge, 16, D)
    b_edge_p = _pad1(b_edge)
    W_node_p = _pad2(W_node)
    b_node_p = _pad1(b_node)
    layers = []
    for (Wm, bm, Wo, bo, gamma, beta, We1, be1, We2, be2) in (
            (Wm0, bm0, Wo0, bo0, gamma0, beta0, We1_0, be1_0, We2_0, be2_0),
            (Wm1, bm1, Wo1, bo1, gamma1, beta1, We1_1, be1_1, We2_1, be2_1)):
        layers.append(dict(
            Wm_h=_pad2(Wm[:H]), Wm_e=_pad2(Wm[H + 1:]),
            bvec=_pad1(bm + Wm[H]),
            Wo=_pad2(Wo), bo=_pad1(bo), gamma=_pad1(gamma), beta=_pad1(beta),
            We1_a=_pad2(We1[:H]), We1_b=_pad2(We1[H:2 * H]), We1_c=_pad2(We1[2 * H:]),
            be1=_pad1(be1), We2=_pad2(We2), be2=_pad1(be2),
        ))
    bc1_p = _pad1(bc1)
    Wc1_p = _pad2(Wc1)
    Wc2_p = _pad2(Wc2)
    bc2_p = _pad1(bc2)
    Wc3_p = _pad2(Wc3)
    bc3_p = _pad1(bc3)

    src2 = jnp.pad(src, (0, EP - E)).reshape(EP // RPG, RPG)
    sb2 = jnp.pad(sb, (0, EP - E)).reshape(EP // RPG, RPG)
    edge_attr_p = jnp.pad(edge_attr, ((0, EP - E), (0, 0)))

    ones = jnp.ones((E,), jnp.float32)
    cnt = _segsum(ones, sb, S)
    ssrc = _segsum(src.astype(jnp.float32), sb, S)
    sdst = _segsum(dst.astype(jnp.float32), sb, S)
    cnt2 = jnp.pad(cnt, (0, SP - S)).reshape(SP // D, D)
    ssrc2 = jnp.pad(ssrc, (0, SP - S)).reshape(SP // D, D)
    sdst2 = jnp.pad(sdst, (0, SP - S)).reshape(SP // D, D)
    nsrc2, ndst2, vm2 = _seg_meta(cnt2, ssrc2, sdst2)
    ndst_flat = ndst2.reshape(SP)
    vmask = vm2.reshape(SP, 1)
    nsrc_flat = nsrc2.reshape(SP)

    h, hWm = _node_embed(x, W_node_p, b_node_p, layers[0]["Wm_h"])
    eattr = _edge_embed(edge_attr_p, W_edge_p, b_edge_p)

    sb_seg = jnp.concatenate(
        [sb, jnp.full((EP - E,), SP - 1, jnp.int32)])

    for li, L in enumerate(layers):
        ne, sides_flat, sideid_arr = _sc_segsum_ne(eattr, sb_seg)
        sides = sides_flat.reshape(2 * NW, D)
        sideids = sideid_arr.reshape(NW, 16)[:, :2].reshape(1, 2 * NW)
        g1 = hWm[nsrc_flat]
        msg, eW1 = _msg_stage(g1, ne, vmask, sides, sideids,
                              L["Wm_e"], L["bvec"], L["We1_b"])
        agg = _segsum(msg, ndst_flat, N)
        if li == 0:
            Wb = layers[1]["Wm_h"]
        else:
            Wb = Wc1_p
        h, hW1, hWm = _node_update(agg, h, L["Wo"], L["bo"], L["gamma"], L["beta"],
                                   L["We1_a"], Wb)
        g2 = _sc_gather(hW1, src2, EP)
        g3 = _sc_gather(eW1, sb2, EP)
        eattr = _edge_mlp(g2, g3, eattr, L["We1_c"], L["be1"], L["We2"], L["be2"])

    out = _head(hWm, bc1_p, Wc2_p, bc2_p, Wc3_p, bc3_p)
    return out[:, :2]
